# Initial kernel scaffold; baseline (speedup 1.0000x reference)
import jax
import jax.numpy as jnp
from jax.experimental import pallas as pl


def kernel(x, Wg, W1, W2):
    raise NotImplementedError("write your pallas kernel here")


# speedup vs baseline: 2.1688x; 2.1688x over previous
import jax
import jax.numpy as jnp
from jax.experimental import pallas as pl
from jax.experimental.pallas import tpu as pltpu

N_TOK = 2048
DIM = 1024
N_EXP = 8
D_FF = 1024


def _moe_dense_body(x_ref, wg_ref, w1_ref, w2_ref, y_ref, dw_ref):
    e = pl.program_id(0)

    @pl.when(e == 0)
    def _router():
        x = x_ref[...]
        logits = jax.lax.dot_general(
            x, wg_ref[...], (((1,), (1,)), ((), ())),
            preferred_element_type=jnp.float32)
        m = jnp.max(logits, axis=1, keepdims=True)
        ex = jnp.exp(logits - m)
        p = ex / jnp.sum(ex, axis=1, keepdims=True)
        iota = jax.lax.broadcasted_iota(jnp.int32, p.shape, 1)
        m1 = jnp.max(p, axis=1, keepdims=True)
        i1 = jnp.min(jnp.where(p == m1, iota, N_EXP), axis=1, keepdims=True)
        p2 = jnp.where(iota == i1, -jnp.inf, p)
        m2 = jnp.max(p2, axis=1, keepdims=True)
        i2 = jnp.min(jnp.where(p2 == m2, iota, N_EXP), axis=1, keepdims=True)
        s = m1 + m2
        dw = jnp.where(iota == i1, m1 / s,
                       jnp.where(iota == i2, m2 / s, 0.0))
        dw_ref[...] = dw
        y_ref[...] = jnp.zeros_like(y_ref)

    x = x_ref[...]
    h = jnp.dot(x, w1_ref[0], preferred_element_type=jnp.float32)
    h = h * (1.0 / (1.0 + jnp.exp(-h)))
    ye = jnp.dot(h, w2_ref[0], preferred_element_type=jnp.float32)
    iota = jax.lax.broadcasted_iota(jnp.int32, (N_TOK, N_EXP), 1)
    w_col = jnp.sum(jnp.where(iota == e, dw_ref[...], 0.0), axis=1,
                    keepdims=True)
    y_ref[...] += w_col * ye


def kernel(x, Wg, W1, W2):
    return pl.pallas_call(
        _moe_dense_body,
        grid=(N_EXP,),
        in_specs=[
            pl.BlockSpec((N_TOK, DIM), lambda e: (0, 0)),
            pl.BlockSpec((N_EXP, DIM), lambda e: (0, 0)),
            pl.BlockSpec((1, DIM, D_FF), lambda e: (e, 0, 0)),
            pl.BlockSpec((1, D_FF, DIM), lambda e: (e, 0, 0)),
        ],
        out_specs=pl.BlockSpec((N_TOK, DIM), lambda e: (0, 0)),
        out_shape=jax.ShapeDtypeStruct((N_TOK, DIM), jnp.float32),
        scratch_shapes=[pltpu.VMEM((N_TOK, N_EXP), jnp.float32)],
    )(x, Wg, W1, W2)
